# baseline (speedup 1.0000x reference)
import math

import jax
import jax.numpy as jnp
from jax import lax
from jax.experimental import pallas as pl
from jax.experimental.pallas import tpu as pltpu

B, S, D, V, H = 2, 2048, 1024, 32000, 16
BLOCK, NA = 16, 32
GAMMA = 7.0
MASK_ID = V - 1
EPS = 1e-6
T = NA * BLOCK
DH = D // H
HALF = DH // 2
ROWS = B * T
SCALE = 1.0 / math.sqrt(DH)
NEG = -1e30

_INTERPRET = False

f32 = jnp.float32
bf16 = jnp.bfloat16


def _cos_sin(pos):
    inv = jnp.exp(jnp.arange(HALF, dtype=f32) * (-math.log(10000.0) / HALF))
    ang = pos.astype(f32)[:, None] * inv[None, :]
    return jnp.cos(ang), jnp.sin(ang)


def _expand(tab):
    return pltpu.repeat(tab, D // HALF, axis=1)


def _rot_weight(w):
    wr = w.reshape(D, H, 2, HALF)
    return jnp.concatenate([-wr[:, :, 1:2, :], wr[:, :, 0:1, :]], axis=2).reshape(D, D)


_TS = 512


def _ctx_kv_body(h0, h1, h2, wfc, wk, wkr, wv, cs, sn, k_out, v_out):
    x0 = h0[0].astype(bf16)
    x1 = h1[0].astype(bf16)
    x2 = h2[0].astype(bf16)
    ctx = jnp.dot(x0, wfc[0:D], preferred_element_type=f32)
    ctx += jnp.dot(x1, wfc[D:2 * D], preferred_element_type=f32)
    ctx += jnp.dot(x2, wfc[2 * D:3 * D], preferred_element_type=f32)
    ctx = ctx.astype(bf16)
    cos = _expand(cs[...])
    sin = _expand(sn[...])
    k = jnp.dot(ctx, wk[...], preferred_element_type=f32)
    kp = jnp.dot(ctx, wkr[...], preferred_element_type=f32)
    k_out[0] = (k * cos + kp * sin).astype(bf16)
    v_out[0] = jnp.dot(ctx, wv[...], preferred_element_type=f32).astype(bf16)


def _ctx_kv(h0, h1, h2, wfc_bf, wk_bf, wkr_bf, wv_bf, cs, sn):
    hspec = pl.BlockSpec((1, _TS, D), lambda b, s: (b, s, 0))
    wspec3 = pl.BlockSpec((3 * D, D), lambda b, s: (0, 0))
    wspec = pl.BlockSpec((D, D), lambda b, s: (0, 0))
    tspec = pl.BlockSpec((_TS, HALF), lambda b, s: (s, 0))
    ospec = pl.BlockSpec((1, _TS, D), lambda b, s: (b, s, 0))
    out = jax.ShapeDtypeStruct((B, S, D), bf16)
    return pl.pallas_call(
        _ctx_kv_body,
        grid=(B, S // _TS),
        in_specs=[hspec, hspec, hspec, wspec3, wspec, wspec, wspec, tspec, tspec],
        out_specs=[ospec, ospec],
        out_shape=[out, out],
        interpret=_INTERPRET,
    )(h0, h1, h2, wfc_bf, wk_bf, wkr_bf, wv_bf, cs, sn)


def _qkv_body(emb, wq, wqr, wk, wkr, wv, cs, sn, q_out, k_out, v_out):
    x = emb[...].astype(bf16)
    cos = _expand(cs[...])
    sin = _expand(sn[...])
    q = jnp.dot(x, wq[...], preferred_element_type=f32)
    qp = jnp.dot(x, wqr[...], preferred_element_type=f32)
    q_out[...] = (q * cos + qp * sin).astype(bf16)
    k = jnp.dot(x, wk[...], preferred_element_type=f32)
    kp = jnp.dot(x, wkr[...], preferred_element_type=f32)
    k_out[...] = (k * cos + kp * sin).astype(bf16)
    v_out[...] = jnp.dot(x, wv[...], preferred_element_type=f32).astype(bf16)


def _qkv(emb, wq_bf, wqr_bf, wk_bf, wkr_bf, wv_bf, cs, sn):
    out = jax.ShapeDtypeStruct((ROWS, D), bf16)
    return pl.pallas_call(
        _qkv_body,
        out_shape=[out, out, out],
        interpret=_INTERPRET,
    )(emb, wq_bf, wqr_bf, wk_bf, wkr_bf, wv_bf, cs, sn)


_HP = 2


def _attn_body(q, kc, vc, kd, vd, anq, out, cb_s):
    h = pl.program_id(1)

    @pl.when(h == 0)
    def _bias():
        ccol = lax.broadcasted_iota(jnp.int32, (T, S), 1).astype(f32)
        cb_s[...] = jnp.where(ccol < anq[0], 0.0, NEG)

    rblk = lax.broadcasted_iota(jnp.int32, (T, T), 0) // BLOCK
    cblk = lax.broadcasted_iota(jnp.int32, (T, T), 1) // BLOCK
    d_bias = jnp.where(rblk == cblk, 0.0, NEG)
    ctx_bias = cb_s[...]
    outs = []
    for i in range(_HP):
        sl = slice(i * DH, (i + 1) * DH)
        qh = q[0][:, sl]
        lc = jax.lax.dot_general(qh, kc[0][:, sl], (((1,), (1,)), ((), ())),
                                 preferred_element_type=f32) * SCALE + ctx_bias
        ld = jax.lax.dot_general(qh, kd[0][:, sl], (((1,), (1,)), ((), ())),
                                 preferred_element_type=f32) * SCALE + d_bias
        m = jnp.maximum(jnp.max(lc, axis=1, keepdims=True),
                        jnp.max(ld, axis=1, keepdims=True))
        pc = jnp.exp(lc - m)
        pd = jnp.exp(ld - m)
        den = jnp.sum(pc, axis=1, keepdims=True) + jnp.sum(pd, axis=1, keepdims=True)
        oh = jnp.dot(pc.astype(bf16), vc[0][:, sl], preferred_element_type=f32)
        oh += jnp.dot(pd.astype(bf16), vd[0][:, sl], preferred_element_type=f32)
        outs.append((oh / den).astype(bf16))
    out[0] = jnp.concatenate(outs, axis=1)


def _attn(q, kc, vc, kd, vd, anq):
    LW = _HP * DH
    dspec = pl.BlockSpec((1, T, LW), lambda b, h: (b, 0, h))
    cspec = pl.BlockSpec((1, S, LW), lambda b, h: (b, 0, h))
    aspec = pl.BlockSpec((1, T, 1), lambda b, h: (b, 0, 0))
    return pl.pallas_call(
        _attn_body,
        grid=(B, H // _HP),
        in_specs=[dspec, cspec, cspec, dspec, dspec, aspec],
        out_specs=dspec,
        out_shape=jax.ShapeDtypeStruct((B, T, D), bf16),
        scratch_shapes=[pltpu.VMEM((T, S), f32)],
        interpret=_INTERPRET,
    )(q, kc, vc, kd, vd, anq)


_FT = 1024
_NF = 4 * D // _FT


def _mlp_body(attn, emb, wo, w1, w2, nw, out, h_s, acc):
    j = pl.program_id(0)

    @pl.when(j == 0)
    def _init():
        h_s[...] = emb[...] + jnp.dot(attn[...], wo[...], preferred_element_type=f32)
        acc[...] = jnp.zeros((ROWS, D), f32)

    hb = h_s[...].astype(bf16)
    a1 = jnp.maximum(jnp.dot(hb, w1[...], preferred_element_type=f32), 0.0)
    acc[...] += jnp.dot(a1.astype(bf16), w2[...], preferred_element_type=f32)

    @pl.when(j == _NF - 1)
    def _fin():
        h2 = h_s[...] + acc[...]
        rms = lax.rsqrt(jnp.mean(h2 * h2, axis=1, keepdims=True) + EPS)
        out[...] = (h2 * rms * nw[...]).astype(bf16)


def _mlp(attn, emb, wo_bf, w1_bf, w2_bf, nw):
    full = pl.BlockSpec((ROWS, D), lambda j: (0, 0))
    wspec = pl.BlockSpec((D, D), lambda j: (0, 0))
    w1spec = pl.BlockSpec((D, _FT), lambda j: (0, j))
    w2spec = pl.BlockSpec((_FT, D), lambda j: (j, 0))
    nwspec = pl.BlockSpec((1, D), lambda j: (0, 0))
    return pl.pallas_call(
        _mlp_body,
        grid=(_NF,),
        in_specs=[full, full, wspec, w1spec, w2spec, nwspec],
        out_specs=full,
        out_shape=jax.ShapeDtypeStruct((ROWS, D), bf16),
        scratch_shapes=[pltpu.VMEM((ROWS, D), f32), pltpu.VMEM((ROWS, D), f32)],
        interpret=_INTERPRET,
    )(attn, emb, wo_bf, w1_bf, w2_bf, nw)


_VT = 1024
_NV = V // _VT


def _lm_body(hn, lmw, wlab, w, valid, loss, acc_o, s_s, bv_s):
    j = pl.program_id(0)

    @pl.when(j == 0)
    def _init():
        s_s[...] = jnp.zeros((ROWS, 1), f32)
        bv_s[...] = jnp.full((ROWS, 1), NEG, f32)

    x = hn[...]
    wt = lmw[...].astype(bf16)
    lg = jax.lax.dot_general(x, wt, (((1,), (1,)), ((), ())),
                             preferred_element_type=f32)
    s_s[...] += jnp.sum(jnp.exp(lg), axis=1, keepdims=True)
    bv_s[...] = jnp.maximum(bv_s[...], jnp.max(lg, axis=1, keepdims=True))

    @pl.when(j == _NV - 1)
    def _fin():
        ll = jnp.sum(x.astype(f32) * wlab[...].astype(f32),
                     axis=1, keepdims=True)
        nll = jnp.log(s_s[...]) - ll
        ww = w[...]
        num_l = jnp.sum(ww * nll)
        den_l = jnp.maximum(jnp.sum(ww), 1e-6)
        match = (ll >= bv_s[...] - 1e-3).astype(f32)
        vv = valid[...]
        num_a = jnp.sum(vv * match)
        den_a = jnp.maximum(jnp.sum(vv), 1.0)
        loss[...] = (num_l / den_l).reshape(1, 1)
        acc_o[...] = (num_a / den_a).reshape(1, 1)


def _lm_loss(hn, lm_head_weight, wlab, w, valid):
    full = pl.BlockSpec((ROWS, D), lambda j: (0, 0))
    wspec = pl.BlockSpec((_VT, D), lambda j: (j, 0))
    cspec = pl.BlockSpec((ROWS, 1), lambda j: (0, 0))
    sspec = pl.BlockSpec((1, 1), lambda j: (0, 0))
    scal = jax.ShapeDtypeStruct((1, 1), f32)
    return pl.pallas_call(
        _lm_body,
        grid=(_NV,),
        in_specs=[full, wspec, full, cspec, cspec],
        out_specs=[sspec, sspec],
        out_shape=[scal, scal],
        scratch_shapes=[pltpu.VMEM((ROWS, 1), f32)] * 2,
        interpret=_INTERPRET,
    )(hn, lm_head_weight, wlab, w, valid)


def kernel(input_ids, hidden_states_0, hidden_states_1, hidden_states_2,
           loss_mask, lm_head_weight, norm_weight, embed, W_fc, Wq, Wk, Wv,
           Wo, W1, W2):
    valid_end = S - BLOCK
    g = jax.random.gumbel(jax.random.key(42), (B, valid_end))
    sc = jnp.where(loss_mask[:, :valid_end] > 0, g, -1e9)
    _, idx = jax.lax.top_k(sc, NA)
    anchors = jnp.sort(idx, axis=-1)
    offsets = jnp.arange(BLOCK)
    all_pos = (anchors[:, :, None] + offsets[None, None, :]).reshape(B, T)
    tokens = jnp.take_along_axis(input_ids, all_pos, axis=1)
    pos_in_block = jnp.arange(T) % BLOCK
    is_anchor = (pos_in_block == 0)[None, :]
    draft_ids = jnp.where(is_anchor, tokens, MASK_ID)
    labels = jnp.where(is_anchor, -100, tokens)
    emb = jnp.take(embed, draft_ids.reshape(ROWS), axis=0)
    wlab = jnp.take(lm_head_weight, jnp.maximum(labels, 0).reshape(ROWS),
                    axis=0).astype(bf16)

    kk = jnp.arange(BLOCK, dtype=f32)
    decay = jnp.where(kk == 0, 0.0, jnp.exp(-(kk - 1.0) / GAMMA))
    valid_col = (labels != -100).reshape(ROWS, 1).astype(f32)
    w_col = decay[pos_in_block][None, :].repeat(B, 0).reshape(ROWS, 1) * valid_col
    anq = jnp.repeat(anchors, BLOCK, axis=1).reshape(B, T, 1).astype(f32)
    ctx_cos, ctx_sin = _cos_sin(jnp.arange(S))
    d_cos, d_sin = _cos_sin(all_pos.reshape(ROWS))

    wfc_bf = W_fc.astype(bf16)
    wq_bf, wqr_bf = Wq.astype(bf16), _rot_weight(Wq).astype(bf16)
    wk_bf, wkr_bf = Wk.astype(bf16), _rot_weight(Wk).astype(bf16)
    wv_bf = Wv.astype(bf16)
    wo_bf, w1_bf, w2_bf = Wo.astype(bf16), W1.astype(bf16), W2.astype(bf16)
    nw = norm_weight.reshape(1, D)

    k_ctx, v_ctx = _ctx_kv(hidden_states_0, hidden_states_1, hidden_states_2,
                           wfc_bf, wk_bf, wkr_bf, wv_bf, ctx_cos, ctx_sin)
    q, k_d, v_d = _qkv(emb, wq_bf, wqr_bf, wk_bf, wkr_bf, wv_bf, d_cos, d_sin)
    attn = _attn(q.reshape(B, T, D), k_ctx, v_ctx,
                 k_d.reshape(B, T, D), v_d.reshape(B, T, D), anq)
    hn = _mlp(attn.reshape(ROWS, D), emb, wo_bf, w1_bf, w2_bf, nw)
    loss, acc = _lm_loss(hn, lm_head_weight, wlab, w_col, valid_col)
    return (loss.reshape(()), acc.reshape(()))


# speedup vs baseline: 1.0344x; 1.0344x over previous
import math

import jax
import jax.numpy as jnp
from jax import lax
from jax.experimental import pallas as pl
from jax.experimental.pallas import tpu as pltpu

B, S, D, V, H = 2, 2048, 1024, 32000, 16
BLOCK, NA = 16, 32
GAMMA = 7.0
MASK_ID = V - 1
EPS = 1e-6
T = NA * BLOCK
DH = D // H
HALF = DH // 2
ROWS = B * T
SCALE = 1.0 / math.sqrt(DH)
LOG2E = math.log2(math.e)
QSCALE = SCALE * LOG2E
NEG = -1e30

_INTERPRET = False

f32 = jnp.float32
bf16 = jnp.bfloat16


def _cos_sin(pos):
    inv = jnp.exp(jnp.arange(HALF, dtype=f32) * (-math.log(10000.0) / HALF))
    ang = pos.astype(f32)[:, None] * inv[None, :]
    return jnp.cos(ang), jnp.sin(ang)


def _expand(tab):
    return pltpu.repeat(tab, D // HALF, axis=1)


def _rot_weight(w):
    wr = w.reshape(D, H, 2, HALF)
    return jnp.concatenate([-wr[:, :, 1:2, :], wr[:, :, 0:1, :]], axis=2).reshape(D, D)


_TS = 512


def _ctx_kv_body(h0, h1, h2, wfc, wk, wkr, wv, cs, sn, k_out, v_out):
    x0 = h0[0].astype(bf16)
    x1 = h1[0].astype(bf16)
    x2 = h2[0].astype(bf16)
    ctx = jnp.dot(x0, wfc[0:D], preferred_element_type=f32)
    ctx += jnp.dot(x1, wfc[D:2 * D], preferred_element_type=f32)
    ctx += jnp.dot(x2, wfc[2 * D:3 * D], preferred_element_type=f32)
    ctx = ctx.astype(bf16)
    cos = _expand(cs[...])
    sin = _expand(sn[...])
    k = jnp.dot(ctx, wk[...], preferred_element_type=f32)
    kp = jnp.dot(ctx, wkr[...], preferred_element_type=f32)
    k_out[0] = (k * cos + kp * sin).astype(bf16)
    v_out[0] = jnp.dot(ctx, wv[...], preferred_element_type=f32).astype(bf16)


def _ctx_kv(h0, h1, h2, wfc_bf, wk_bf, wkr_bf, wv_bf, cs, sn):
    hspec = pl.BlockSpec((1, _TS, D), lambda b, s: (b, s, 0))
    wspec3 = pl.BlockSpec((3 * D, D), lambda b, s: (0, 0))
    wspec = pl.BlockSpec((D, D), lambda b, s: (0, 0))
    tspec = pl.BlockSpec((_TS, HALF), lambda b, s: (s, 0))
    ospec = pl.BlockSpec((1, _TS, D), lambda b, s: (b, s, 0))
    out = jax.ShapeDtypeStruct((B, S, D), bf16)
    return pl.pallas_call(
        _ctx_kv_body,
        grid=(B, S // _TS),
        in_specs=[hspec, hspec, hspec, wspec3, wspec, wspec, wspec, tspec, tspec],
        out_specs=[ospec, ospec],
        out_shape=[out, out],
        interpret=_INTERPRET,
    )(h0, h1, h2, wfc_bf, wk_bf, wkr_bf, wv_bf, cs, sn)


def _qkv_body(emb, wq, wqr, wk, wkr, wv, cs, sn, q_out, k_out, v_out):
    x = emb[...].astype(bf16)
    cos = _expand(cs[...])
    sin = _expand(sn[...])
    q = jnp.dot(x, wq[...], preferred_element_type=f32)
    qp = jnp.dot(x, wqr[...], preferred_element_type=f32)
    q_out[...] = ((q * cos + qp * sin) * QSCALE).astype(bf16)
    k = jnp.dot(x, wk[...], preferred_element_type=f32)
    kp = jnp.dot(x, wkr[...], preferred_element_type=f32)
    k_out[...] = (k * cos + kp * sin).astype(bf16)
    v_out[...] = jnp.dot(x, wv[...], preferred_element_type=f32).astype(bf16)


def _qkv(emb, wq_bf, wqr_bf, wk_bf, wkr_bf, wv_bf, cs, sn):
    out = jax.ShapeDtypeStruct((ROWS, D), bf16)
    return pl.pallas_call(
        _qkv_body,
        out_shape=[out, out, out],
        interpret=_INTERPRET,
    )(emb, wq_bf, wqr_bf, wk_bf, wkr_bf, wv_bf, cs, sn)


_HP = 2


def _attn_body(q, kc, vc, kd, vd, anq, out, cb_s, db_s):
    h = pl.program_id(1)

    @pl.when(h == 0)
    def _bias():
        ccol = lax.broadcasted_iota(jnp.int32, (T, S), 1).astype(f32)
        cb_s[...] = jnp.where(ccol < anq[0], 0.0, NEG)
        rblk = lax.broadcasted_iota(jnp.int32, (T, T), 0) // BLOCK
        cblk = lax.broadcasted_iota(jnp.int32, (T, T), 1) // BLOCK
        db_s[...] = jnp.where(rblk == cblk, 0.0, NEG)

    ctx_bias = cb_s[...]
    d_bias = db_s[...]
    outs = []
    for i in range(_HP):
        sl = slice(i * DH, (i + 1) * DH)
        qh = q[0][:, sl]
        pc = jnp.exp2(jax.lax.dot_general(
            qh, kc[0][:, sl], (((1,), (1,)), ((), ())),
            preferred_element_type=f32) + ctx_bias)
        pd = jnp.exp2(jax.lax.dot_general(
            qh, kd[0][:, sl], (((1,), (1,)), ((), ())),
            preferred_element_type=f32) + d_bias)
        den = jnp.sum(pc, axis=1, keepdims=True) + jnp.sum(pd, axis=1, keepdims=True)
        oh = jnp.dot(pc.astype(bf16), vc[0][:, sl], preferred_element_type=f32)
        oh += jnp.dot(pd.astype(bf16), vd[0][:, sl], preferred_element_type=f32)
        outs.append((oh / den).astype(bf16))
    out[0] = jnp.concatenate(outs, axis=1)


def _attn(q, kc, vc, kd, vd, anq):
    LW = _HP * DH
    dspec = pl.BlockSpec((1, T, LW), lambda b, h: (b, 0, h))
    cspec = pl.BlockSpec((1, S, LW), lambda b, h: (b, 0, h))
    aspec = pl.BlockSpec((1, T, 1), lambda b, h: (b, 0, 0))
    return pl.pallas_call(
        _attn_body,
        grid=(B, H // _HP),
        in_specs=[dspec, cspec, cspec, dspec, dspec, aspec],
        out_specs=dspec,
        out_shape=jax.ShapeDtypeStruct((B, T, D), bf16),
        scratch_shapes=[pltpu.VMEM((T, S), f32), pltpu.VMEM((T, T), f32)],
        interpret=_INTERPRET,
    )(q, kc, vc, kd, vd, anq)


_FT = 1024
_NF = 4 * D // _FT


def _mlp_body(attn, emb, wo, w1, w2, nw, out, h_s, acc):
    j = pl.program_id(0)

    @pl.when(j == 0)
    def _init():
        h_s[...] = emb[...] + jnp.dot(attn[...], wo[...], preferred_element_type=f32)
        acc[...] = jnp.zeros((ROWS, D), f32)

    hb = h_s[...].astype(bf16)
    a1 = jnp.maximum(jnp.dot(hb, w1[...], preferred_element_type=f32), 0.0)
    acc[...] += jnp.dot(a1.astype(bf16), w2[...], preferred_element_type=f32)

    @pl.when(j == _NF - 1)
    def _fin():
        h2 = h_s[...] + acc[...]
        rms = lax.rsqrt(jnp.mean(h2 * h2, axis=1, keepdims=True) + EPS)
        out[...] = (h2 * rms * nw[...]).astype(bf16)


def _mlp(attn, emb, wo_bf, w1_bf, w2_bf, nw):
    full = pl.BlockSpec((ROWS, D), lambda j: (0, 0))
    wspec = pl.BlockSpec((D, D), lambda j: (0, 0))
    w1spec = pl.BlockSpec((D, _FT), lambda j: (0, j))
    w2spec = pl.BlockSpec((_FT, D), lambda j: (j, 0))
    nwspec = pl.BlockSpec((1, D), lambda j: (0, 0))
    return pl.pallas_call(
        _mlp_body,
        grid=(_NF,),
        in_specs=[full, full, wspec, w1spec, w2spec, nwspec],
        out_specs=full,
        out_shape=jax.ShapeDtypeStruct((ROWS, D), bf16),
        scratch_shapes=[pltpu.VMEM((ROWS, D), f32), pltpu.VMEM((ROWS, D), f32)],
        interpret=_INTERPRET,
    )(attn, emb, wo_bf, w1_bf, w2_bf, nw)


_VT = 1280
_NV = V // _VT
assert _NV * _VT == V


def _lm_body(hn, hn2, lmw, wlab, w, valid, loss, acc_o, s_s, bv_s):
    j = pl.program_id(0)

    @pl.when(j == 0)
    def _init():
        s_s[...] = jnp.zeros((ROWS, 1), f32)
        bv_s[...] = jnp.full((ROWS, 1), NEG, f32)

    x2 = hn2[...]
    wt = lmw[...].astype(bf16)
    lg2 = jax.lax.dot_general(x2, wt, (((1,), (1,)), ((), ())),
                              preferred_element_type=f32)
    s_s[...] += jnp.sum(jnp.exp2(lg2), axis=1, keepdims=True)
    bv_s[...] = jnp.maximum(bv_s[...], jnp.max(lg2, axis=1, keepdims=True))

    @pl.when(j == _NV - 1)
    def _fin():
        x = hn[...]
        wl = wlab[...].astype(f32)
        ll = jnp.sum(x.astype(f32) * wl, axis=1, keepdims=True)
        ll2 = jnp.sum(x2.astype(f32) * wl, axis=1, keepdims=True)
        nll = jnp.log(s_s[...]) - ll
        ww = w[...]
        num_l = jnp.sum(ww * nll)
        den_l = jnp.maximum(jnp.sum(ww), 1e-6)
        match = (ll2 >= bv_s[...] - 1e-3).astype(f32)
        vv = valid[...]
        num_a = jnp.sum(vv * match)
        den_a = jnp.maximum(jnp.sum(vv), 1.0)
        loss[...] = (num_l / den_l).reshape(1, 1)
        acc_o[...] = (num_a / den_a).reshape(1, 1)


def _lm_loss(hn, lm_head_weight, wlab, w, valid):
    full = pl.BlockSpec((ROWS, D), lambda j: (0, 0))
    wspec = pl.BlockSpec((_VT, D), lambda j: (j, 0))
    cspec = pl.BlockSpec((ROWS, 1), lambda j: (0, 0))
    sspec = pl.BlockSpec((1, 1), lambda j: (0, 0))
    scal = jax.ShapeDtypeStruct((1, 1), f32)
    hn2 = (hn.astype(f32) * LOG2E).astype(bf16)
    return pl.pallas_call(
        _lm_body,
        grid=(_NV,),
        in_specs=[full, full, wspec, full, cspec, cspec],
        out_specs=[sspec, sspec],
        out_shape=[scal, scal],
        scratch_shapes=[pltpu.VMEM((ROWS, 1), f32)] * 2,
        interpret=_INTERPRET,
    )(hn, hn2, lm_head_weight, wlab, w, valid)


def kernel(input_ids, hidden_states_0, hidden_states_1, hidden_states_2,
           loss_mask, lm_head_weight, norm_weight, embed, W_fc, Wq, Wk, Wv,
           Wo, W1, W2):
    valid_end = S - BLOCK
    g = jax.random.gumbel(jax.random.key(42), (B, valid_end))
    sc = jnp.where(loss_mask[:, :valid_end] > 0, g, -1e9)
    _, idx = jax.lax.top_k(sc, NA)
    anchors = jnp.sort(idx, axis=-1)
    offsets = jnp.arange(BLOCK)
    all_pos = (anchors[:, :, None] + offsets[None, None, :]).reshape(B, T)
    tokens = jnp.take_along_axis(input_ids, all_pos, axis=1)
    pos_in_block = jnp.arange(T) % BLOCK
    is_anchor = (pos_in_block == 0)[None, :]
    draft_ids = jnp.where(is_anchor, tokens, MASK_ID)
    labels = jnp.where(is_anchor, -100, tokens)
    emb = jnp.take(embed, draft_ids.reshape(ROWS), axis=0)
    wlab = jnp.take(lm_head_weight, jnp.maximum(labels, 0).reshape(ROWS),
                    axis=0).astype(bf16)

    kk = jnp.arange(BLOCK, dtype=f32)
    decay = jnp.where(kk == 0, 0.0, jnp.exp(-(kk - 1.0) / GAMMA))
    valid_col = (labels != -100).reshape(ROWS, 1).astype(f32)
    w_col = decay[pos_in_block][None, :].repeat(B, 0).reshape(ROWS, 1) * valid_col
    anq = jnp.repeat(anchors, BLOCK, axis=1).reshape(B, T, 1).astype(f32)
    ctx_cos, ctx_sin = _cos_sin(jnp.arange(S))
    d_cos, d_sin = _cos_sin(all_pos.reshape(ROWS))

    wfc_bf = W_fc.astype(bf16)
    wq_bf, wqr_bf = Wq.astype(bf16), _rot_weight(Wq).astype(bf16)
    wk_bf, wkr_bf = Wk.astype(bf16), _rot_weight(Wk).astype(bf16)
    wv_bf = Wv.astype(bf16)
    wo_bf, w1_bf, w2_bf = Wo.astype(bf16), W1.astype(bf16), W2.astype(bf16)
    nw = norm_weight.reshape(1, D)

    k_ctx, v_ctx = _ctx_kv(hidden_states_0, hidden_states_1, hidden_states_2,
                           wfc_bf, wk_bf, wkr_bf, wv_bf, ctx_cos, ctx_sin)
    q, k_d, v_d = _qkv(emb, wq_bf, wqr_bf, wk_bf, wkr_bf, wv_bf, d_cos, d_sin)
    attn = _attn(q.reshape(B, T, D), k_ctx, v_ctx,
                 k_d.reshape(B, T, D), v_d.reshape(B, T, D), anq)
    hn = _mlp(attn.reshape(ROWS, D), emb, wo_bf, w1_bf, w2_bf, nw)
    loss, acc = _lm_loss(hn, lm_head_weight, wlab, w_col, valid_col)
    return (loss.reshape(()), acc.reshape(()))


# speedup vs baseline: 1.1830x; 1.1437x over previous
import math

import functools

import jax
import jax.numpy as jnp
from jax import lax
from jax.experimental import pallas as pl
from jax.experimental.pallas import tpu as pltpu
from jax.experimental.pallas import tpu_sc as plsc

B, S, D, V, H = 2, 2048, 1024, 32000, 16
BLOCK, NA = 16, 32
GAMMA = 7.0
MASK_ID = V - 1
EPS = 1e-6
T = NA * BLOCK
DH = D // H
HALF = DH // 2
ROWS = B * T
SCALE = 1.0 / math.sqrt(DH)
LOG2E = math.log2(math.e)
QSCALE = SCALE * LOG2E
NEG = -1e30

_INTERPRET = False

f32 = jnp.float32
bf16 = jnp.bfloat16


def _cos_sin(pos):
    inv = jnp.exp(jnp.arange(HALF, dtype=f32) * (-math.log(10000.0) / HALF))
    ang = pos.astype(f32)[:, None] * inv[None, :]
    return jnp.cos(ang), jnp.sin(ang)


def _expand(tab):
    return pltpu.repeat(tab, D // HALF, axis=1)


def _rot_weight(w):
    wr = w.reshape(D, H, 2, HALF)
    return jnp.concatenate([-wr[:, :, 1:2, :], wr[:, :, 0:1, :]], axis=2).reshape(D, D)


_NW = 32
_RPW = ROWS // _NW


def _sc_gather(embed_hbm, lmw_hbm, dids, labs):
    mesh = plsc.VectorSubcoreMesh(core_axis_name="c", subcore_axis_name="s")

    @functools.partial(
        pl.kernel, mesh=mesh,
        out_type=[jax.ShapeDtypeStruct((ROWS, D), f32),
                  jax.ShapeDtypeStruct((ROWS, D), f32)],
        scratch_types=[pltpu.VMEM((_RPW,), jnp.int32),
                       pltpu.VMEM((_RPW, D), f32),
                       pltpu.VMEM((_RPW,), jnp.int32),
                       pltpu.VMEM((_RPW, D), f32),
                       pltpu.SemaphoreType.DMA,
                       pltpu.SemaphoreType.DMA],
    )
    def k(emb_t, lmw_t, dids_h, labs_h, emb_o, wlab_o,
          idx1, rows1, idx2, rows2, sem1, sem2):
        wid = lax.axis_index("s") * 2 + lax.axis_index("c")
        base = wid * _RPW
        pltpu.sync_copy(dids_h.at[pl.ds(base, _RPW)], idx1)
        pltpu.sync_copy(labs_h.at[pl.ds(base, _RPW)], idx2)
        c1 = pltpu.async_copy(emb_t.at[idx1], rows1, sem1)
        c2 = pltpu.async_copy(lmw_t.at[idx2], rows2, sem2)
        c1.wait()
        c2.wait()
        pltpu.sync_copy(rows1, emb_o.at[pl.ds(base, _RPW)])
        pltpu.sync_copy(rows2, wlab_o.at[pl.ds(base, _RPW)])

    return k(embed_hbm, lmw_hbm, dids, labs)


_TS = 512


def _ctx_kv_body(h0, h1, h2, wfc, wk, wkr, wv, cs, sn, k_out, v_out):
    x0 = h0[0].astype(bf16)
    x1 = h1[0].astype(bf16)
    x2 = h2[0].astype(bf16)
    ctx = jnp.dot(x0, wfc[0:D].astype(bf16), preferred_element_type=f32)
    ctx += jnp.dot(x1, wfc[D:2 * D].astype(bf16), preferred_element_type=f32)
    ctx += jnp.dot(x2, wfc[2 * D:3 * D].astype(bf16), preferred_element_type=f32)
    ctx = ctx.astype(bf16)
    cos = _expand(cs[...])
    sin = _expand(sn[...])
    k = jnp.dot(ctx, wk[...].astype(bf16), preferred_element_type=f32)
    kp = jnp.dot(ctx, wkr[...], preferred_element_type=f32)
    k_out[0] = (k * cos + kp * sin).astype(bf16)
    v_out[0] = jnp.dot(ctx, wv[...].astype(bf16), preferred_element_type=f32).astype(bf16)


def _ctx_kv(h0, h1, h2, wfc_bf, wk_bf, wkr_bf, wv_bf, cs, sn):
    hspec = pl.BlockSpec((1, _TS, D), lambda b, s: (b, s, 0))
    wspec3 = pl.BlockSpec((3 * D, D), lambda b, s: (0, 0))
    wspec = pl.BlockSpec((D, D), lambda b, s: (0, 0))
    tspec = pl.BlockSpec((_TS, HALF), lambda b, s: (s, 0))
    ospec = pl.BlockSpec((1, _TS, D), lambda b, s: (b, s, 0))
    out = jax.ShapeDtypeStruct((B, S, D), bf16)
    return pl.pallas_call(
        _ctx_kv_body,
        grid=(B, S // _TS),
        in_specs=[hspec, hspec, hspec, wspec3, wspec, wspec, wspec, tspec, tspec],
        out_specs=[ospec, ospec],
        out_shape=[out, out],
        interpret=_INTERPRET,
    )(h0, h1, h2, wfc_bf, wk_bf, wkr_bf, wv_bf, cs, sn)


def _qkv_body(emb, wq, wqr, wk, wkr, wv, cs, sn, q_out, k_out, v_out):
    x = emb[...].astype(bf16)
    cos = _expand(cs[...])
    sin = _expand(sn[...])
    q = jnp.dot(x, wq[...].astype(bf16), preferred_element_type=f32)
    qp = jnp.dot(x, wqr[...], preferred_element_type=f32)
    q_out[...] = ((q * cos + qp * sin) * QSCALE).astype(bf16)
    k = jnp.dot(x, wk[...].astype(bf16), preferred_element_type=f32)
    kp = jnp.dot(x, wkr[...], preferred_element_type=f32)
    k_out[...] = (k * cos + kp * sin).astype(bf16)
    v_out[...] = jnp.dot(x, wv[...].astype(bf16), preferred_element_type=f32).astype(bf16)


def _qkv(emb, wq_bf, wqr_bf, wk_bf, wkr_bf, wv_bf, cs, sn):
    out = jax.ShapeDtypeStruct((ROWS, D), bf16)
    return pl.pallas_call(
        _qkv_body,
        out_shape=[out, out, out],
        interpret=_INTERPRET,
    )(emb, wq_bf, wqr_bf, wk_bf, wkr_bf, wv_bf, cs, sn)


_HP = 2


def _attn_body(q, kc, vc, kd, vd, anq, out, cb_s, db_s):
    h = pl.program_id(1)

    @pl.when(h == 0)
    def _bias():
        ccol = lax.broadcasted_iota(jnp.int32, (T, S), 1).astype(f32)
        cb_s[...] = jnp.where(ccol < anq[0], 0.0, NEG)
        rblk = lax.broadcasted_iota(jnp.int32, (T, T), 0) // BLOCK
        cblk = lax.broadcasted_iota(jnp.int32, (T, T), 1) // BLOCK
        db_s[...] = jnp.where(rblk == cblk, 0.0, NEG)

    ctx_bias = cb_s[...]
    d_bias = db_s[...]
    outs = []
    for i in range(_HP):
        sl = slice(i * DH, (i + 1) * DH)
        qh = q[0][:, sl]
        pc = jnp.exp2(jax.lax.dot_general(
            qh, kc[0][:, sl], (((1,), (1,)), ((), ())),
            preferred_element_type=f32) + ctx_bias)
        pd = jnp.exp2(jax.lax.dot_general(
            qh, kd[0][:, sl], (((1,), (1,)), ((), ())),
            preferred_element_type=f32) + d_bias)
        den = jnp.sum(pc, axis=1, keepdims=True) + jnp.sum(pd, axis=1, keepdims=True)
        oh = jnp.dot(pc.astype(bf16), vc[0][:, sl], preferred_element_type=f32)
        oh += jnp.dot(pd.astype(bf16), vd[0][:, sl], preferred_element_type=f32)
        outs.append((oh / den).astype(bf16))
    out[0] = jnp.concatenate(outs, axis=1)


def _attn(q, kc, vc, kd, vd, anq):
    LW = _HP * DH
    dspec = pl.BlockSpec((1, T, LW), lambda b, h: (b, 0, h))
    cspec = pl.BlockSpec((1, S, LW), lambda b, h: (b, 0, h))
    aspec = pl.BlockSpec((1, T, 1), lambda b, h: (b, 0, 0))
    return pl.pallas_call(
        _attn_body,
        grid=(B, H // _HP),
        in_specs=[dspec, cspec, cspec, dspec, dspec, aspec],
        out_specs=dspec,
        out_shape=jax.ShapeDtypeStruct((B, T, D), bf16),
        scratch_shapes=[pltpu.VMEM((T, S), f32), pltpu.VMEM((T, T), f32)],
        interpret=_INTERPRET,
    )(q, kc, vc, kd, vd, anq)


_FT = 1024
_NF = 4 * D // _FT


def _mlp_body(attn, emb, wo, w1, w2, nw, out, h_s, acc):
    j = pl.program_id(0)

    @pl.when(j == 0)
    def _init():
        h_s[...] = emb[...] + jnp.dot(attn[...], wo[...].astype(bf16),
                                      preferred_element_type=f32)
        acc[...] = jnp.zeros((ROWS, D), f32)

    hb = h_s[...].astype(bf16)
    a1 = jnp.maximum(jnp.dot(hb, w1[...].astype(bf16),
                             preferred_element_type=f32), 0.0)
    acc[...] += jnp.dot(a1.astype(bf16), w2[...].astype(bf16),
                        preferred_element_type=f32)

    @pl.when(j == _NF - 1)
    def _fin():
        h2 = h_s[...] + acc[...]
        rms = lax.rsqrt(jnp.mean(h2 * h2, axis=1, keepdims=True) + EPS)
        out[...] = (h2 * rms * nw[...]).astype(bf16)


def _mlp(attn, emb, wo_bf, w1_bf, w2_bf, nw):
    full = pl.BlockSpec((ROWS, D), lambda j: (0, 0))
    wspec = pl.BlockSpec((D, D), lambda j: (0, 0))
    w1spec = pl.BlockSpec((D, _FT), lambda j: (0, j))
    w2spec = pl.BlockSpec((_FT, D), lambda j: (j, 0))
    nwspec = pl.BlockSpec((1, D), lambda j: (0, 0))
    return pl.pallas_call(
        _mlp_body,
        grid=(_NF,),
        in_specs=[full, full, wspec, w1spec, w2spec, nwspec],
        out_specs=full,
        out_shape=jax.ShapeDtypeStruct((ROWS, D), bf16),
        scratch_shapes=[pltpu.VMEM((ROWS, D), f32), pltpu.VMEM((ROWS, D), f32)],
        interpret=_INTERPRET,
    )(attn, emb, wo_bf, w1_bf, w2_bf, nw)


_VT = 1280
_NV = V // _VT
assert _NV * _VT == V


def _lm_body(hn, hn2, lmw, wlab, w, valid, loss, acc_o, s_s, bv_s):
    j = pl.program_id(0)

    @pl.when(j == 0)
    def _init():
        s_s[...] = jnp.zeros((ROWS, 1), f32)
        bv_s[...] = jnp.full((ROWS, 1), NEG, f32)

    x2 = hn2[...]
    wt = lmw[...].astype(bf16)
    lg2 = jax.lax.dot_general(x2, wt, (((1,), (1,)), ((), ())),
                              preferred_element_type=f32)
    s_s[...] += jnp.sum(jnp.exp2(lg2), axis=1, keepdims=True)
    bv_s[...] = jnp.maximum(bv_s[...], jnp.max(lg2, axis=1, keepdims=True))

    @pl.when(j == _NV - 1)
    def _fin():
        x = hn[...]
        wl = wlab[...]
        ll = jnp.sum(x.astype(f32) * wl, axis=1, keepdims=True)
        ll2 = jnp.sum(x2.astype(f32) * wl, axis=1, keepdims=True)
        nll = jnp.log(s_s[...]) - ll
        ww = w[...]
        num_l = jnp.sum(ww * nll)
        den_l = jnp.maximum(jnp.sum(ww), 1e-6)
        match = (ll2 >= bv_s[...] - 1e-3).astype(f32)
        vv = valid[...]
        num_a = jnp.sum(vv * match)
        den_a = jnp.maximum(jnp.sum(vv), 1.0)
        loss[...] = (num_l / den_l).reshape(1, 1)
        acc_o[...] = (num_a / den_a).reshape(1, 1)


def _lm_loss(hn, lm_head_weight, wlab, w, valid):
    full = pl.BlockSpec((ROWS, D), lambda j: (0, 0))
    wspec = pl.BlockSpec((_VT, D), lambda j: (j, 0))
    cspec = pl.BlockSpec((ROWS, 1), lambda j: (0, 0))
    sspec = pl.BlockSpec((1, 1), lambda j: (0, 0))
    scal = jax.ShapeDtypeStruct((1, 1), f32)
    hn2 = (hn.astype(f32) * LOG2E).astype(bf16)
    return pl.pallas_call(
        _lm_body,
        grid=(_NV,),
        in_specs=[full, full, wspec, full, cspec, cspec],
        out_specs=[sspec, sspec],
        out_shape=[scal, scal],
        scratch_shapes=[pltpu.VMEM((ROWS, 1), f32)] * 2,
        interpret=_INTERPRET,
    )(hn, hn2, lm_head_weight, wlab, w, valid)


def kernel(input_ids, hidden_states_0, hidden_states_1, hidden_states_2,
           loss_mask, lm_head_weight, norm_weight, embed, W_fc, Wq, Wk, Wv,
           Wo, W1, W2):
    valid_end = S - BLOCK
    g = jax.random.gumbel(jax.random.key(42), (B, valid_end))
    sc = jnp.where(loss_mask[:, :valid_end] > 0, g, -1e9)
    _, idx = jax.lax.top_k(sc, NA)
    anchors = jnp.sort(idx, axis=-1)
    offsets = jnp.arange(BLOCK)
    all_pos = (anchors[:, :, None] + offsets[None, None, :]).reshape(B, T)
    tokens = jnp.take_along_axis(input_ids, all_pos, axis=1)
    pos_in_block = jnp.arange(T) % BLOCK
    is_anchor = (pos_in_block == 0)[None, :]
    draft_ids = jnp.where(is_anchor, tokens, MASK_ID)
    labels = jnp.where(is_anchor, -100, tokens)
    emb, wlab = _sc_gather(embed, lm_head_weight, draft_ids.reshape(ROWS),
                           jnp.maximum(labels, 0).reshape(ROWS))

    kk = jnp.arange(BLOCK, dtype=f32)
    decay = jnp.where(kk == 0, 0.0, jnp.exp(-(kk - 1.0) / GAMMA))
    valid_col = (labels != -100).reshape(ROWS, 1).astype(f32)
    w_col = decay[pos_in_block][None, :].repeat(B, 0).reshape(ROWS, 1) * valid_col
    anq = jnp.repeat(anchors, BLOCK, axis=1).reshape(B, T, 1).astype(f32)
    ctx_cos, ctx_sin = _cos_sin(jnp.arange(S))
    d_cos, d_sin = _cos_sin(all_pos.reshape(ROWS))

    wqr_bf = _rot_weight(Wq).astype(bf16)
    wkr_bf = _rot_weight(Wk).astype(bf16)
    nw = norm_weight.reshape(1, D)

    k_ctx, v_ctx = _ctx_kv(hidden_states_0, hidden_states_1, hidden_states_2,
                           W_fc, Wk, wkr_bf, Wv, ctx_cos, ctx_sin)
    q, k_d, v_d = _qkv(emb, Wq, wqr_bf, Wk, wkr_bf, Wv, d_cos, d_sin)
    attn = _attn(q.reshape(B, T, D), k_ctx, v_ctx,
                 k_d.reshape(B, T, D), v_d.reshape(B, T, D), anq)
    hn = _mlp(attn.reshape(ROWS, D), emb, Wo, W1, W2, nw)
    loss, acc = _lm_loss(hn, lm_head_weight, wlab, w_col, valid_col)
    return (loss.reshape(()), acc.reshape(()))


# speedup vs baseline: 1.2335x; 1.0427x over previous
import math

import functools

import jax
import jax.numpy as jnp
from jax import lax
from jax.experimental import pallas as pl
from jax.experimental.pallas import tpu as pltpu
from jax.experimental.pallas import tpu_sc as plsc

B, S, D, V, H = 2, 2048, 1024, 32000, 16
BLOCK, NA = 16, 32
GAMMA = 7.0
MASK_ID = V - 1
EPS = 1e-6
T = NA * BLOCK
DH = D // H
HALF = DH // 2
ROWS = B * T
SCALE = 1.0 / math.sqrt(DH)
LOG2E = math.log2(math.e)
QSCALE = SCALE * LOG2E
NEG = -1e30

_INTERPRET = False

f32 = jnp.float32
bf16 = jnp.bfloat16


def _cos_sin(pos):
    inv = jnp.exp(jnp.arange(HALF, dtype=f32) * (-math.log(10000.0) / HALF))
    ang = pos.astype(f32)[:, None] * inv[None, :]
    return jnp.cos(ang), jnp.sin(ang)


def _expand(tab):
    return pltpu.repeat(tab, D // HALF, axis=1)


def _rot_weight(w):
    wr = w.reshape(D, H, 2, HALF)
    return jnp.concatenate([-wr[:, :, 1:2, :], wr[:, :, 0:1, :]], axis=2).reshape(D, D)


_NW = 32
_RPW = ROWS // _NW


def _sc_gather(embed_hbm, lmw_hbm, dids, labs):
    mesh = plsc.VectorSubcoreMesh(core_axis_name="c", subcore_axis_name="s")

    @functools.partial(
        pl.kernel, mesh=mesh,
        out_type=[jax.ShapeDtypeStruct((ROWS, D), f32),
                  jax.ShapeDtypeStruct((ROWS, D), f32)],
        scratch_types=[pltpu.VMEM((_RPW,), jnp.int32),
                       pltpu.VMEM((_RPW, D), f32),
                       pltpu.VMEM((_RPW,), jnp.int32),
                       pltpu.VMEM((_RPW, D), f32),
                       pltpu.SemaphoreType.DMA,
                       pltpu.SemaphoreType.DMA],
    )
    def k(emb_t, lmw_t, dids_h, labs_h, emb_o, wlab_o,
          idx1, rows1, idx2, rows2, sem1, sem2):
        wid = lax.axis_index("s") * 2 + lax.axis_index("c")
        base = wid * _RPW
        pltpu.sync_copy(dids_h.at[pl.ds(base, _RPW)], idx1)
        pltpu.sync_copy(labs_h.at[pl.ds(base, _RPW)], idx2)
        c1 = pltpu.async_copy(emb_t.at[idx1], rows1, sem1)
        c2 = pltpu.async_copy(lmw_t.at[idx2], rows2, sem2)
        c1.wait()
        c2.wait()
        pltpu.sync_copy(rows1, emb_o.at[pl.ds(base, _RPW)])
        pltpu.sync_copy(rows2, wlab_o.at[pl.ds(base, _RPW)])

    return k(embed_hbm, lmw_hbm, dids, labs)


_TS = 512


def _ctx_kv_body(h0, h1, h2, wfc, wk, wkr, wv, cs, sn, k_out, v_out):
    x0 = h0[0].astype(bf16)
    x1 = h1[0].astype(bf16)
    x2 = h2[0].astype(bf16)
    ctx = jnp.dot(x0, wfc[0:D].astype(bf16), preferred_element_type=f32)
    ctx += jnp.dot(x1, wfc[D:2 * D].astype(bf16), preferred_element_type=f32)
    ctx += jnp.dot(x2, wfc[2 * D:3 * D].astype(bf16), preferred_element_type=f32)
    ctx = ctx.astype(bf16)
    cos = _expand(cs[...])
    sin = _expand(sn[...])
    k = jnp.dot(ctx, wk[...].astype(bf16), preferred_element_type=f32)
    kp = jnp.dot(ctx, wkr[...], preferred_element_type=f32)
    k_out[0] = (k * cos + kp * sin).astype(bf16)
    v_out[0] = jnp.dot(ctx, wv[...].astype(bf16), preferred_element_type=f32).astype(bf16)


def _ctx_kv(h0, h1, h2, wfc_bf, wk_bf, wkr_bf, wv_bf, cs, sn):
    hspec = pl.BlockSpec((1, _TS, D), lambda b, s: (b, s, 0))
    wspec3 = pl.BlockSpec((3 * D, D), lambda b, s: (0, 0))
    wspec = pl.BlockSpec((D, D), lambda b, s: (0, 0))
    tspec = pl.BlockSpec((_TS, HALF), lambda b, s: (s, 0))
    ospec = pl.BlockSpec((1, _TS, D), lambda b, s: (b, s, 0))
    out = jax.ShapeDtypeStruct((B, S, D), bf16)
    return pl.pallas_call(
        _ctx_kv_body,
        grid=(B, S // _TS),
        in_specs=[hspec, hspec, hspec, wspec3, wspec, wspec, wspec, tspec, tspec],
        out_specs=[ospec, ospec],
        out_shape=[out, out],
        interpret=_INTERPRET,
    )(h0, h1, h2, wfc_bf, wk_bf, wkr_bf, wv_bf, cs, sn)


def _qkv_body(emb, wq, wqr, wk, wkr, wv, cs, sn, q_out, k_out, v_out):
    x = emb[...].astype(bf16)
    cos = _expand(cs[...])
    sin = _expand(sn[...])
    q = jnp.dot(x, wq[...].astype(bf16), preferred_element_type=f32)
    qp = jnp.dot(x, wqr[...], preferred_element_type=f32)
    q_out[...] = ((q * cos + qp * sin) * QSCALE).astype(bf16)
    k = jnp.dot(x, wk[...].astype(bf16), preferred_element_type=f32)
    kp = jnp.dot(x, wkr[...], preferred_element_type=f32)
    k_out[...] = (k * cos + kp * sin).astype(bf16)
    v_out[...] = jnp.dot(x, wv[...].astype(bf16), preferred_element_type=f32).astype(bf16)


def _qkv(emb, wq_bf, wqr_bf, wk_bf, wkr_bf, wv_bf, cs, sn):
    out = jax.ShapeDtypeStruct((ROWS, D), bf16)
    return pl.pallas_call(
        _qkv_body,
        out_shape=[out, out, out],
        interpret=_INTERPRET,
    )(emb, wq_bf, wqr_bf, wk_bf, wkr_bf, wv_bf, cs, sn)


_HP = 4


def _attn_body(q, kc, vc, kd, vd, anq, out, cb_s, db_s):
    h = pl.program_id(1)

    @pl.when(h == 0)
    def _bias():
        ccol = lax.broadcasted_iota(jnp.int32, (T, S), 1).astype(f32)
        cb_s[...] = jnp.where(ccol < anq[0], 0.0, NEG)
        rblk = lax.broadcasted_iota(jnp.int32, (T, T), 0) // BLOCK
        cblk = lax.broadcasted_iota(jnp.int32, (T, T), 1) // BLOCK
        db_s[...] = jnp.where(rblk == cblk, 0.0, NEG)

    ctx_bias = cb_s[...]
    d_bias = db_s[...]
    outs = []
    for i in range(_HP):
        sl = slice(i * DH, (i + 1) * DH)
        qh = q[0][:, sl]
        pc = jnp.exp2(jax.lax.dot_general(
            qh, kc[0][:, sl], (((1,), (1,)), ((), ())),
            preferred_element_type=f32) + ctx_bias)
        pd = jnp.exp2(jax.lax.dot_general(
            qh, kd[0][:, sl], (((1,), (1,)), ((), ())),
            preferred_element_type=f32) + d_bias)
        den = jnp.sum(pc, axis=1, keepdims=True) + jnp.sum(pd, axis=1, keepdims=True)
        oh = jnp.dot(pc.astype(bf16), vc[0][:, sl], preferred_element_type=f32)
        oh += jnp.dot(pd.astype(bf16), vd[0][:, sl], preferred_element_type=f32)
        outs.append((oh / den).astype(bf16))
    out[0] = jnp.concatenate(outs, axis=1)


def _attn(q, kc, vc, kd, vd, anq):
    LW = _HP * DH
    dspec = pl.BlockSpec((1, T, LW), lambda b, h: (b, 0, h))
    cspec = pl.BlockSpec((1, S, LW), lambda b, h: (b, 0, h))
    aspec = pl.BlockSpec((1, T, 1), lambda b, h: (b, 0, 0))
    return pl.pallas_call(
        _attn_body,
        grid=(B, H // _HP),
        in_specs=[dspec, cspec, cspec, dspec, dspec, aspec],
        out_specs=dspec,
        out_shape=jax.ShapeDtypeStruct((B, T, D), bf16),
        scratch_shapes=[pltpu.VMEM((T, S), f32), pltpu.VMEM((T, T), f32)],
        interpret=_INTERPRET,
    )(q, kc, vc, kd, vd, anq)


_FT = 1024
_NF = 4 * D // _FT


def _mlp_body(attn, emb, wo, w1, w2, nw, out, h_s, acc):
    j = pl.program_id(0)

    @pl.when(j == 0)
    def _init():
        h_s[...] = emb[...] + jnp.dot(attn[...], wo[...].astype(bf16),
                                      preferred_element_type=f32)
        acc[...] = jnp.zeros((ROWS, D), f32)

    hb = h_s[...].astype(bf16)
    a1 = jnp.maximum(jnp.dot(hb, w1[...].astype(bf16),
                             preferred_element_type=f32), 0.0)
    acc[...] += jnp.dot(a1.astype(bf16), w2[...].astype(bf16),
                        preferred_element_type=f32)

    @pl.when(j == _NF - 1)
    def _fin():
        h2 = h_s[...] + acc[...]
        rms = lax.rsqrt(jnp.mean(h2 * h2, axis=1, keepdims=True) + EPS)
        out[...] = (h2 * rms * nw[...]).astype(bf16)


def _mlp(attn, emb, wo_bf, w1_bf, w2_bf, nw):
    full = pl.BlockSpec((ROWS, D), lambda j: (0, 0))
    wspec = pl.BlockSpec((D, D), lambda j: (0, 0))
    w1spec = pl.BlockSpec((D, _FT), lambda j: (0, j))
    w2spec = pl.BlockSpec((_FT, D), lambda j: (j, 0))
    nwspec = pl.BlockSpec((1, D), lambda j: (0, 0))
    return pl.pallas_call(
        _mlp_body,
        grid=(_NF,),
        in_specs=[full, full, wspec, w1spec, w2spec, nwspec],
        out_specs=full,
        out_shape=jax.ShapeDtypeStruct((ROWS, D), bf16),
        scratch_shapes=[pltpu.VMEM((ROWS, D), f32), pltpu.VMEM((ROWS, D), f32)],
        interpret=_INTERPRET,
    )(attn, emb, wo_bf, w1_bf, w2_bf, nw)


_VT = 1280
_NV = V // _VT
assert _NV * _VT == V


def _lm_body(hn, hn2, lmw, wlab, w, valid, loss, acc_o, s_s, bv_s):
    j = pl.program_id(0)

    @pl.when(j == 0)
    def _init():
        s_s[...] = jnp.zeros((ROWS, 1), f32)
        bv_s[...] = jnp.full((ROWS, 1), NEG, f32)

    x2 = hn2[...]
    wt = lmw[...].astype(bf16)
    lg2 = jax.lax.dot_general(x2, wt, (((1,), (1,)), ((), ())),
                              preferred_element_type=f32)
    s_s[...] += jnp.sum(jnp.exp2(lg2), axis=1, keepdims=True)
    bv_s[...] = jnp.maximum(bv_s[...], jnp.max(lg2, axis=1, keepdims=True))

    @pl.when(j == _NV - 1)
    def _fin():
        x = hn[...]
        wl = wlab[...]
        ll = jnp.sum(x.astype(f32) * wl, axis=1, keepdims=True)
        ll2 = jnp.sum(x2.astype(f32) * wl, axis=1, keepdims=True)
        nll = jnp.log(s_s[...]) - ll
        ww = w[...]
        num_l = jnp.sum(ww * nll)
        den_l = jnp.maximum(jnp.sum(ww), 1e-6)
        match = (ll2 >= bv_s[...] - 1e-3).astype(f32)
        vv = valid[...]
        num_a = jnp.sum(vv * match)
        den_a = jnp.maximum(jnp.sum(vv), 1.0)
        loss[...] = (num_l / den_l).reshape(1, 1)
        acc_o[...] = (num_a / den_a).reshape(1, 1)


def _lm_loss(hn, lm_head_weight, wlab, w, valid):
    full = pl.BlockSpec((ROWS, D), lambda j: (0, 0))
    wspec = pl.BlockSpec((_VT, D), lambda j: (j, 0))
    cspec = pl.BlockSpec((ROWS, 1), lambda j: (0, 0))
    sspec = pl.BlockSpec((1, 1), lambda j: (0, 0))
    scal = jax.ShapeDtypeStruct((1, 1), f32)
    hn2 = (hn.astype(f32) * LOG2E).astype(bf16)
    return pl.pallas_call(
        _lm_body,
        grid=(_NV,),
        in_specs=[full, full, wspec, full, cspec, cspec],
        out_specs=[sspec, sspec],
        out_shape=[scal, scal],
        scratch_shapes=[pltpu.VMEM((ROWS, 1), f32)] * 2,
        interpret=_INTERPRET,
    )(hn, hn2, lm_head_weight, wlab, w, valid)


def kernel(input_ids, hidden_states_0, hidden_states_1, hidden_states_2,
           loss_mask, lm_head_weight, norm_weight, embed, W_fc, Wq, Wk, Wv,
           Wo, W1, W2):
    valid_end = S - BLOCK
    g = jax.random.gumbel(jax.random.key(42), (B, valid_end))
    sc = jnp.where(loss_mask[:, :valid_end] > 0, g, -1e9)
    _, idx = jax.lax.top_k(sc, NA)
    anchors = jnp.sort(idx, axis=-1)
    offsets = jnp.arange(BLOCK)
    all_pos = (anchors[:, :, None] + offsets[None, None, :]).reshape(B, T)
    tokens = jnp.take_along_axis(input_ids, all_pos, axis=1)
    pos_in_block = jnp.arange(T) % BLOCK
    is_anchor = (pos_in_block == 0)[None, :]
    draft_ids = jnp.where(is_anchor, tokens, MASK_ID)
    labels = jnp.where(is_anchor, -100, tokens)
    emb, wlab = _sc_gather(embed, lm_head_weight, draft_ids.reshape(ROWS),
                           jnp.maximum(labels, 0).reshape(ROWS))

    kk = jnp.arange(BLOCK, dtype=f32)
    decay = jnp.where(kk == 0, 0.0, jnp.exp(-(kk - 1.0) / GAMMA))
    valid_col = (labels != -100).reshape(ROWS, 1).astype(f32)
    w_col = decay[pos_in_block][None, :].repeat(B, 0).reshape(ROWS, 1) * valid_col
    anq = jnp.repeat(anchors, BLOCK, axis=1).reshape(B, T, 1).astype(f32)
    ctx_cos, ctx_sin = _cos_sin(jnp.arange(S))
    d_cos, d_sin = _cos_sin(all_pos.reshape(ROWS))

    wqr_bf = _rot_weight(Wq).astype(bf16)
    wkr_bf = _rot_weight(Wk).astype(bf16)
    nw = norm_weight.reshape(1, D)

    k_ctx, v_ctx = _ctx_kv(hidden_states_0, hidden_states_1, hidden_states_2,
                           W_fc, Wk, wkr_bf, Wv, ctx_cos, ctx_sin)
    q, k_d, v_d = _qkv(emb, Wq, wqr_bf, Wk, wkr_bf, Wv, d_cos, d_sin)
    attn = _attn(q.reshape(B, T, D), k_ctx, v_ctx,
                 k_d.reshape(B, T, D), v_d.reshape(B, T, D), anq)
    hn = _mlp(attn.reshape(ROWS, D), emb, Wo, W1, W2, nw)
    loss, acc = _lm_loss(hn, lm_head_weight, wlab, w_col, valid_col)
    return (loss.reshape(()), acc.reshape(()))
